# baseline (speedup 1.0000x reference)
import functools

import jax
import jax.numpy as jnp
from jax import lax
from jax.experimental import pallas as pl
from jax.experimental.pallas import tpu as pltpu
from jax.experimental.pallas import tpu_sc as plsc

S = 4
B = 512
NN = 64
E = 128
ME = 100000
MARGIN = 1.0

NC = 2
NS = 16
NW = NC * NS

C = 128
RING = 4

P = 4
BP = B // P
TPW = S * BP // NW
NCHUNK = (S * BP * NN) // (NW * C)


def _sc_gather_body(ent, relt, sidx_h, ridx_h, nidx_h,
                    h_out, t_out, rel_out, neg_out,
                    sidx_v, ridx_v, nidx_v, sbuf, rbuf, nbufs,
                    ssem, rsem, nsem0, nsem1, nsem2, nsem3):
    nsems = (nsem0, nsem1, nsem2, nsem3)
    wid = lax.axis_index("s") * NC + lax.axis_index("c")

    pltpu.sync_copy(nidx_h.at[wid], nidx_v)
    pltpu.sync_copy(sidx_h.at[wid], sidx_v)
    pltpu.sync_copy(ridx_h.at[wid], ridx_v)

    for b in range(RING):
        pltpu.async_copy(ent.at[nidx_v.at[b]], nbufs.at[b], nsems[b])

    pltpu.async_copy(ent.at[sidx_v], sbuf, ssem)
    pltpu.async_copy(relt.at[ridx_v], rbuf, rsem)

    nbase = wid * (NCHUNK * C)

    def step(k, b):
        pltpu.make_async_copy(ent.at[nidx_v.at[k]], nbufs.at[b], nsems[b]).wait()
        pltpu.sync_copy(nbufs.at[b], neg_out.at[pl.ds(nbase + k * C, C)])

    def outer(i, _):
        for b in range(RING):
            k = i * RING + b
            step(k, b)
            pltpu.async_copy(ent.at[nidx_v.at[k + RING]], nbufs.at[b], nsems[b])
        return ()

    lax.fori_loop(0, NCHUNK // RING - 1, outer, ())
    for b in range(RING):
        step(NCHUNK - RING + b, b)

    pltpu.make_async_copy(ent.at[sidx_v], sbuf, ssem).wait()
    sb = wid * TPW
    pltpu.sync_copy(sbuf.at[pl.ds(0, TPW)], h_out.at[pl.ds(sb, TPW)])
    pltpu.sync_copy(sbuf.at[pl.ds(TPW, TPW)], t_out.at[pl.ds(sb, TPW)])
    pltpu.make_async_copy(relt.at[ridx_v], rbuf, rsem).wait()
    pltpu.sync_copy(rbuf, rel_out.at[pl.ds(sb, TPW)])


@jax.jit
def _sc_gather(ent, relt, sidx, ridx, nidx):
    mesh = plsc.VectorSubcoreMesh(core_axis_name="c", subcore_axis_name="s")
    f = pl.kernel(
        _sc_gather_body,
        out_type=[
            jax.ShapeDtypeStruct((S * BP, E), jnp.float32),
            jax.ShapeDtypeStruct((S * BP, E), jnp.float32),
            jax.ShapeDtypeStruct((S * BP, E), jnp.float32),
            jax.ShapeDtypeStruct((S * BP * NN, E), jnp.float32),
        ],
        mesh=mesh,
        scratch_types=[
            pltpu.VMEM((2 * TPW,), jnp.int32),
            pltpu.VMEM((TPW,), jnp.int32),
            pltpu.VMEM((NCHUNK, C), jnp.int32),
            pltpu.VMEM((2 * TPW, E), jnp.float32),
            pltpu.VMEM((TPW, E), jnp.float32),
            pltpu.VMEM((RING, C, E), jnp.float32),
            pltpu.SemaphoreType.DMA,
            pltpu.SemaphoreType.DMA,
            pltpu.SemaphoreType.DMA,
            pltpu.SemaphoreType.DMA,
            pltpu.SemaphoreType.DMA,
            pltpu.SemaphoreType.DMA,
        ],
    )
    return f(ent, relt, sidx, ridx, nidx)


TB = 32
NTILE = BP // TB


def _softplus(x):
    return jnp.maximum(x, 0.0) + jnp.log1p(jnp.exp(-jnp.abs(x)))


def _score_body(h_ref, r_ref, t_ref, w_ref, neg_ref, pos_ref, ns_ref, loss_ref):
    i = pl.program_id(0)
    hr = h_ref[...] + r_ref[...]
    d = hr - t_ref[...]
    pos = MARGIN - jnp.sqrt(jnp.sum(d * d, axis=-1) + 1e-12)
    pos_ref[...] = pos

    neg = neg_ref[...]
    nh = jnp.sum(hr * hr, axis=-1)
    nn = jnp.sum(neg * neg, axis=-1)
    dots = lax.dot_general(hr, neg, (((2,), (2,)), ((0,), (0,))),
                           preferred_element_type=jnp.float32)
    ns_sq = nh[:, :, None] + nn[:, None, :] - 2.0 * dots
    ns = MARGIN - jnp.sqrt(jnp.maximum(ns_sq, 0.0) + 1e-12)
    ns_ref[...] = ns

    w = w_ref[...]
    acc = jnp.sum(w * _softplus(-pos))
    acc += jnp.sum(w * jnp.mean(_softplus(ns), axis=-1))

    @pl.when(i == 0)
    def _():
        loss_ref[...] = jnp.zeros((1, 1), jnp.float32)
    loss_ref[...] += 0.5 * acc.reshape(1, 1)


@jax.jit
def _score(h4, r4, t4, w4, neg3):
    grid = (NTILE,)
    pos, ns, loss = pl.pallas_call(
        _score_body,
        grid=grid,
        in_specs=[
            pl.BlockSpec((TB, S, E), lambda i: (i, 0, 0)),
            pl.BlockSpec((TB, S, E), lambda i: (i, 0, 0)),
            pl.BlockSpec((TB, S, E), lambda i: (i, 0, 0)),
            pl.BlockSpec((TB, S), lambda i: (i, 0)),
            pl.BlockSpec((TB, S * NN, E), lambda i: (i, 0, 0)),
        ],
        out_specs=[
            pl.BlockSpec((TB, S), lambda i: (i, 0)),
            pl.BlockSpec((TB, S, S * NN), lambda i: (i, 0, 0)),
            pl.BlockSpec((1, 1), lambda i: (0, 0)),
        ],
        out_shape=[
            jax.ShapeDtypeStruct((BP, S), jnp.float32),
            jax.ShapeDtypeStruct((BP, S, S * NN), jnp.float32),
            jax.ShapeDtypeStruct((1, 1), jnp.float32),
        ],
        compiler_params=pltpu.CompilerParams(
            dimension_semantics=("arbitrary",),
        ),
    )(h4, r4, t4, w4, neg3)
    return pos, ns, loss


def kernel(head, relation, tail, negative, triple_weight, entity_embedding,
           relation_embedding):
    head = head[0]
    relation = relation[0]
    tail = tail[0]
    negative = negative[0]
    w = triple_weight[0]

    ent = entity_embedding.reshape(S * ME, E)

    offs = (jnp.arange(S, dtype=jnp.int32) * ME)
    neg_flat = negative.reshape(S, B * NN)
    idx_in = jnp.concatenate([tail, neg_flat], axis=1)
    chunk = (B + B * NN) // S
    g = idx_in.reshape(S, S, chunk) + offs[:, None, None]
    out_idx = g.transpose(1, 0, 2).reshape(S, B + B * NN)
    t_idx = out_idx[:, :B].transpose(1, 0).reshape(-1)
    neg_idx = out_idx[:, B:].reshape(S, B, NN).transpose(1, 0, 2).reshape(-1)
    h_idx = (head + offs[:, None]).transpose(1, 0).reshape(-1)

    h_p = h_idx.reshape(P, NW, TPW)
    t_p = t_idx.reshape(P, NW, TPW)
    r_p = relation.transpose(1, 0).reshape(P, NW, TPW)
    n_p = neg_idx.reshape(P, NW, NCHUNK, C)
    w_p = w.transpose(1, 0).reshape(P, BP, S)

    gathered = [
        _sc_gather(ent, relation_embedding,
                   jnp.concatenate([h_p[p], t_p[p]], axis=1),
                   r_p[p], n_p[p])
        for p in range(P)
    ]

    poss, nss, losses = [], [], []
    for p in range(P):
        h_rows, t_rows, rel_rows, neg_rows = gathered[p]
        pos, ns, loss = _score(h_rows.reshape(BP, S, E),
                               rel_rows.reshape(BP, S, E),
                               t_rows.reshape(BP, S, E),
                               w_p[p],
                               neg_rows.reshape(BP, S * NN, E))
        poss.append(pos)
        nss.append(ns)
        losses.append(loss[0, 0])

    pos = jnp.concatenate(poss, axis=0)
    ns = jnp.concatenate(nss, axis=0)
    positive_score = pos.transpose(1, 0).reshape(S * B)
    negative_score = ns.transpose(1, 0, 2).reshape(S * B, S * NN)
    return (sum(losses), positive_score, negative_score)


# speedup vs baseline: 1.0344x; 1.0344x over previous
import functools

import jax
import jax.numpy as jnp
from jax import lax
from jax.experimental import pallas as pl
from jax.experimental.pallas import tpu as pltpu
from jax.experimental.pallas import tpu_sc as plsc

S = 4
B = 512
NN = 64
E = 128
ME = 100000
MARGIN = 1.0

NC = 2
NS = 16
NW = NC * NS

C = 128
RING = 4

P = 2
BP = B // P
TPW = S * BP // NW
NCHUNK = (S * BP * NN) // (NW * C)


def _sc_gather_body(ent, relt, sidx_h, ridx_h, nidx_h,
                    h_out, t_out, rel_out, neg_out,
                    sidx_v, ridx_v, nidx_v, sbuf, rbuf, nbufs,
                    ssem, rsem, nsem0, nsem1, nsem2, nsem3):
    nsems = (nsem0, nsem1, nsem2, nsem3)
    wid = lax.axis_index("s") * NC + lax.axis_index("c")

    pltpu.sync_copy(nidx_h.at[wid], nidx_v)
    pltpu.sync_copy(sidx_h.at[wid], sidx_v)
    pltpu.sync_copy(ridx_h.at[wid], ridx_v)

    for b in range(RING):
        pltpu.async_copy(ent.at[nidx_v.at[b]], nbufs.at[b], nsems[b])

    pltpu.async_copy(ent.at[sidx_v], sbuf, ssem)
    pltpu.async_copy(relt.at[ridx_v], rbuf, rsem)

    nbase = wid * (NCHUNK * C)

    def step(k, b):
        pltpu.make_async_copy(ent.at[nidx_v.at[k]], nbufs.at[b], nsems[b]).wait()
        pltpu.sync_copy(nbufs.at[b], neg_out.at[pl.ds(nbase + k * C, C)])

    def outer(i, _):
        for b in range(RING):
            k = i * RING + b
            step(k, b)
            pltpu.async_copy(ent.at[nidx_v.at[k + RING]], nbufs.at[b], nsems[b])
        return ()

    lax.fori_loop(0, NCHUNK // RING - 1, outer, ())
    for b in range(RING):
        step(NCHUNK - RING + b, b)

    pltpu.make_async_copy(ent.at[sidx_v], sbuf, ssem).wait()
    sb = wid * TPW
    pltpu.sync_copy(sbuf.at[pl.ds(0, TPW)], h_out.at[pl.ds(sb, TPW)])
    pltpu.sync_copy(sbuf.at[pl.ds(TPW, TPW)], t_out.at[pl.ds(sb, TPW)])
    pltpu.make_async_copy(relt.at[ridx_v], rbuf, rsem).wait()
    pltpu.sync_copy(rbuf, rel_out.at[pl.ds(sb, TPW)])


@jax.jit
def _sc_gather(ent, relt, sidx, ridx, nidx):
    mesh = plsc.VectorSubcoreMesh(core_axis_name="c", subcore_axis_name="s")
    f = pl.kernel(
        _sc_gather_body,
        out_type=[
            jax.ShapeDtypeStruct((S * BP, E), jnp.float32),
            jax.ShapeDtypeStruct((S * BP, E), jnp.float32),
            jax.ShapeDtypeStruct((S * BP, E), jnp.float32),
            jax.ShapeDtypeStruct((S * BP * NN, E), jnp.float32),
        ],
        mesh=mesh,
        scratch_types=[
            pltpu.VMEM((2 * TPW,), jnp.int32),
            pltpu.VMEM((TPW,), jnp.int32),
            pltpu.VMEM((NCHUNK, C), jnp.int32),
            pltpu.VMEM((2 * TPW, E), jnp.float32),
            pltpu.VMEM((TPW, E), jnp.float32),
            pltpu.VMEM((RING, C, E), jnp.float32),
            pltpu.SemaphoreType.DMA,
            pltpu.SemaphoreType.DMA,
            pltpu.SemaphoreType.DMA,
            pltpu.SemaphoreType.DMA,
            pltpu.SemaphoreType.DMA,
            pltpu.SemaphoreType.DMA,
        ],
    )
    return f(ent, relt, sidx, ridx, nidx)


TB = 32
NTILE = BP // TB


def _softplus(x):
    return jnp.maximum(x, 0.0) + jnp.log1p(jnp.exp(-jnp.abs(x)))


def _score_body(h_ref, r_ref, t_ref, w_ref, neg_ref, pos_ref, ns_ref, loss_ref):
    i = pl.program_id(0)
    hr = h_ref[...] + r_ref[...]
    d = hr - t_ref[...]
    pos = MARGIN - jnp.sqrt(jnp.sum(d * d, axis=-1) + 1e-12)
    pos_ref[...] = pos

    neg = neg_ref[...]
    nh = jnp.sum(hr * hr, axis=-1)
    nn = jnp.sum(neg * neg, axis=-1)
    dots = lax.dot_general(hr, neg, (((2,), (2,)), ((0,), (0,))),
                           preferred_element_type=jnp.float32)
    ns_sq = nh[:, :, None] + nn[:, None, :] - 2.0 * dots
    ns = MARGIN - jnp.sqrt(jnp.maximum(ns_sq, 0.0) + 1e-12)
    ns_ref[...] = ns.transpose(1, 0, 2)

    w = w_ref[...]
    acc = jnp.sum(w * _softplus(-pos))
    acc += jnp.sum(w * jnp.mean(_softplus(ns), axis=-1))

    @pl.when(i == 0)
    def _():
        loss_ref[...] = jnp.zeros((1, 1), jnp.float32)
    loss_ref[...] += 0.5 * acc.reshape(1, 1)


@jax.jit
def _score(h4, r4, t4, w4, neg3):
    grid = (NTILE,)
    pos, ns, loss = pl.pallas_call(
        _score_body,
        grid=grid,
        in_specs=[
            pl.BlockSpec((TB, S, E), lambda i: (i, 0, 0)),
            pl.BlockSpec((TB, S, E), lambda i: (i, 0, 0)),
            pl.BlockSpec((TB, S, E), lambda i: (i, 0, 0)),
            pl.BlockSpec((TB, S), lambda i: (i, 0)),
            pl.BlockSpec((TB, S * NN, E), lambda i: (i, 0, 0)),
        ],
        out_specs=[
            pl.BlockSpec((TB, S), lambda i: (i, 0)),
            pl.BlockSpec((S, TB, S * NN), lambda i: (0, i, 0)),
            pl.BlockSpec((1, 1), lambda i: (0, 0)),
        ],
        out_shape=[
            jax.ShapeDtypeStruct((BP, S), jnp.float32),
            jax.ShapeDtypeStruct((S, BP, S * NN), jnp.float32),
            jax.ShapeDtypeStruct((1, 1), jnp.float32),
        ],
        compiler_params=pltpu.CompilerParams(
            dimension_semantics=("arbitrary",),
        ),
    )(h4, r4, t4, w4, neg3)
    return pos, ns, loss


def kernel(head, relation, tail, negative, triple_weight, entity_embedding,
           relation_embedding):
    head = head[0]
    relation = relation[0]
    tail = tail[0]
    negative = negative[0]
    w = triple_weight[0]

    ent = entity_embedding.reshape(S * ME, E)

    offs = (jnp.arange(S, dtype=jnp.int32) * ME)
    neg_flat = negative.reshape(S, B * NN)
    idx_in = jnp.concatenate([tail, neg_flat], axis=1)
    chunk = (B + B * NN) // S
    g = idx_in.reshape(S, S, chunk) + offs[:, None, None]
    out_idx = g.transpose(1, 0, 2).reshape(S, B + B * NN)
    t_idx = out_idx[:, :B].transpose(1, 0).reshape(-1)
    neg_idx = out_idx[:, B:].reshape(S, B, NN).transpose(1, 0, 2).reshape(-1)
    h_idx = (head + offs[:, None]).transpose(1, 0).reshape(-1)

    h_p = h_idx.reshape(P, NW, TPW)
    t_p = t_idx.reshape(P, NW, TPW)
    r_p = relation.transpose(1, 0).reshape(P, NW, TPW)
    n_p = neg_idx.reshape(P, NW, NCHUNK, C)
    w_p = w.transpose(1, 0).reshape(P, BP, S)

    gathered = [
        _sc_gather(ent, relation_embedding,
                   jnp.concatenate([h_p[p], t_p[p]], axis=1),
                   r_p[p], n_p[p])
        for p in range(P)
    ]

    poss, nss, losses = [], [], []
    for p in range(P):
        h_rows, t_rows, rel_rows, neg_rows = gathered[p]
        pos, ns, loss = _score(h_rows.reshape(BP, S, E),
                               rel_rows.reshape(BP, S, E),
                               t_rows.reshape(BP, S, E),
                               w_p[p],
                               neg_rows.reshape(BP, S * NN, E))
        poss.append(pos)
        nss.append(ns)
        losses.append(loss[0, 0])

    pos = jnp.concatenate(poss, axis=0)
    ns = jnp.concatenate(nss, axis=1)
    positive_score = pos.transpose(1, 0).reshape(S * B)
    negative_score = ns.reshape(S * B, S * NN)
    return (sum(losses), positive_score, negative_score)


# speedup vs baseline: 1.0477x; 1.0129x over previous
import functools

import jax
import jax.numpy as jnp
from jax import lax
from jax.experimental import pallas as pl
from jax.experimental.pallas import tpu as pltpu
from jax.experimental.pallas import tpu_sc as plsc

S = 4
B = 512
NN = 64
E = 128
ME = 100000
MARGIN = 1.0

NC = 2
NS = 16
NW = NC * NS

C = 128
RING = 4

P = 2
BP = B // P
TPW = S * BP // NW
NCHUNK = (S * BP * NN) // (NW * C)


def _sc_gather_body(ent, relt, sidx_h, ridx_h, nidx_h,
                    h_out, t_out, rel_out, neg_out,
                    sidx_v, ridx_v, nidx_v, sbuf, rbuf, nbufs,
                    ssem, rsem, nsem0, nsem1, nsem2, nsem3):
    nsems = (nsem0, nsem1, nsem2, nsem3)
    wid = lax.axis_index("s") * NC + lax.axis_index("c")

    pltpu.sync_copy(nidx_h.at[wid], nidx_v)
    pltpu.sync_copy(sidx_h.at[wid], sidx_v)
    pltpu.sync_copy(ridx_h.at[wid], ridx_v)

    for b in range(RING):
        pltpu.async_copy(ent.at[nidx_v.at[b]], nbufs.at[b], nsems[b])

    pltpu.async_copy(ent.at[sidx_v], sbuf, ssem)
    pltpu.async_copy(relt.at[ridx_v], rbuf, rsem)

    nbase = wid * (NCHUNK * C)

    def step(k, b):
        pltpu.make_async_copy(ent.at[nidx_v.at[k]], nbufs.at[b], nsems[b]).wait()
        pltpu.sync_copy(nbufs.at[b], neg_out.at[pl.ds(nbase + k * C, C)])

    def outer(i, _):
        for b in range(RING):
            k = i * RING + b
            step(k, b)
            pltpu.async_copy(ent.at[nidx_v.at[k + RING]], nbufs.at[b], nsems[b])
        return ()

    lax.fori_loop(0, NCHUNK // RING - 1, outer, ())
    for b in range(RING):
        step(NCHUNK - RING + b, b)

    pltpu.make_async_copy(ent.at[sidx_v], sbuf, ssem).wait()
    sb = wid * TPW
    pltpu.sync_copy(sbuf.at[pl.ds(0, TPW)], h_out.at[pl.ds(sb, TPW)])
    pltpu.sync_copy(sbuf.at[pl.ds(TPW, TPW)], t_out.at[pl.ds(sb, TPW)])
    pltpu.make_async_copy(relt.at[ridx_v], rbuf, rsem).wait()
    pltpu.sync_copy(rbuf, rel_out.at[pl.ds(sb, TPW)])


@jax.jit
def _sc_gather(ent, relt, sidx, ridx, nidx):
    mesh = plsc.VectorSubcoreMesh(core_axis_name="c", subcore_axis_name="s")
    f = pl.kernel(
        _sc_gather_body,
        out_type=[
            jax.ShapeDtypeStruct((S * BP, E), jnp.float32),
            jax.ShapeDtypeStruct((S * BP, E), jnp.float32),
            jax.ShapeDtypeStruct((S * BP, E), jnp.float32),
            jax.ShapeDtypeStruct((S * BP * NN, E), jnp.float32),
        ],
        mesh=mesh,
        scratch_types=[
            pltpu.VMEM((2 * TPW,), jnp.int32),
            pltpu.VMEM((TPW,), jnp.int32),
            pltpu.VMEM((NCHUNK, C), jnp.int32),
            pltpu.VMEM((2 * TPW, E), jnp.float32),
            pltpu.VMEM((TPW, E), jnp.float32),
            pltpu.VMEM((RING, C, E), jnp.float32),
            pltpu.SemaphoreType.DMA,
            pltpu.SemaphoreType.DMA,
            pltpu.SemaphoreType.DMA,
            pltpu.SemaphoreType.DMA,
            pltpu.SemaphoreType.DMA,
            pltpu.SemaphoreType.DMA,
        ],
    )
    return f(ent, relt, sidx, ridx, nidx)


TB = 64
NTILE = BP // TB


def _softplus(x):
    return jnp.maximum(x, 0.0) + jnp.log1p(jnp.exp(-jnp.abs(x)))


def _score_body(h_ref, r_ref, t_ref, w_ref, neg_ref, pos_ref, ns_ref, loss_ref):
    i = pl.program_id(0)
    hr = h_ref[...] + r_ref[...]
    d = hr - t_ref[...]
    pos = MARGIN - jnp.sqrt(jnp.sum(d * d, axis=-1) + 1e-12)
    pos_ref[...] = pos

    neg = neg_ref[...]
    nh = jnp.sum(hr * hr, axis=-1)
    nn = jnp.sum(neg * neg, axis=-1)
    dots = lax.dot_general(hr, neg, (((2,), (2,)), ((0,), (0,))),
                           preferred_element_type=jnp.float32)
    ns_sq = nh[:, :, None] + nn[:, None, :] - 2.0 * dots
    ns = MARGIN - jnp.sqrt(jnp.maximum(ns_sq, 0.0) + 1e-12)
    ns_ref[...] = ns.transpose(1, 0, 2)

    w = w_ref[...]
    acc = jnp.sum(w * _softplus(-pos))
    acc += jnp.sum(w * jnp.mean(_softplus(ns), axis=-1))

    @pl.when(i == 0)
    def _():
        loss_ref[...] = jnp.zeros((1, 1), jnp.float32)
    loss_ref[...] += 0.5 * acc.reshape(1, 1)


@jax.jit
def _score(h4, r4, t4, w4, neg3):
    grid = (NTILE,)
    pos, ns, loss = pl.pallas_call(
        _score_body,
        grid=grid,
        in_specs=[
            pl.BlockSpec((TB, S, E), lambda i: (i, 0, 0)),
            pl.BlockSpec((TB, S, E), lambda i: (i, 0, 0)),
            pl.BlockSpec((TB, S, E), lambda i: (i, 0, 0)),
            pl.BlockSpec((TB, S), lambda i: (i, 0)),
            pl.BlockSpec((TB, S * NN, E), lambda i: (i, 0, 0)),
        ],
        out_specs=[
            pl.BlockSpec((TB, S), lambda i: (i, 0)),
            pl.BlockSpec((S, TB, S * NN), lambda i: (0, i, 0)),
            pl.BlockSpec((1, 1), lambda i: (0, 0)),
        ],
        out_shape=[
            jax.ShapeDtypeStruct((BP, S), jnp.float32),
            jax.ShapeDtypeStruct((S, BP, S * NN), jnp.float32),
            jax.ShapeDtypeStruct((1, 1), jnp.float32),
        ],
        compiler_params=pltpu.CompilerParams(
            dimension_semantics=("arbitrary",),
        ),
    )(h4, r4, t4, w4, neg3)
    return pos, ns, loss


def kernel(head, relation, tail, negative, triple_weight, entity_embedding,
           relation_embedding):
    head = head[0]
    relation = relation[0]
    tail = tail[0]
    negative = negative[0]
    w = triple_weight[0]

    ent = entity_embedding.reshape(S * ME, E)

    offs = (jnp.arange(S, dtype=jnp.int32) * ME)
    neg_flat = negative.reshape(S, B * NN)
    idx_in = jnp.concatenate([tail, neg_flat], axis=1)
    chunk = (B + B * NN) // S
    g = idx_in.reshape(S, S, chunk) + offs[:, None, None]
    out_idx = g.transpose(1, 0, 2).reshape(S, B + B * NN)
    t_idx = out_idx[:, :B].transpose(1, 0).reshape(-1)
    neg_idx = out_idx[:, B:].reshape(S, B, NN).transpose(1, 0, 2).reshape(-1)
    h_idx = (head + offs[:, None]).transpose(1, 0).reshape(-1)

    h_p = h_idx.reshape(P, NW, TPW)
    t_p = t_idx.reshape(P, NW, TPW)
    r_p = relation.transpose(1, 0).reshape(P, NW, TPW)
    n_p = neg_idx.reshape(P, NW, NCHUNK, C)
    w_p = w.transpose(1, 0).reshape(P, BP, S)

    gathered = [
        _sc_gather(ent, relation_embedding,
                   jnp.concatenate([h_p[p], t_p[p]], axis=1),
                   r_p[p], n_p[p])
        for p in range(P)
    ]

    poss, nss, losses = [], [], []
    for p in range(P):
        h_rows, t_rows, rel_rows, neg_rows = gathered[p]
        pos, ns, loss = _score(h_rows.reshape(BP, S, E),
                               rel_rows.reshape(BP, S, E),
                               t_rows.reshape(BP, S, E),
                               w_p[p],
                               neg_rows.reshape(BP, S * NN, E))
        poss.append(pos)
        nss.append(ns)
        losses.append(loss[0, 0])

    pos = jnp.concatenate(poss, axis=0)
    ns = jnp.concatenate(nss, axis=1)
    positive_score = pos.transpose(1, 0).reshape(S * B)
    negative_score = ns.reshape(S * B, S * NN)
    return (sum(losses), positive_score, negative_score)
